# Initial kernel scaffold; baseline (speedup 1.0000x reference)
import jax
import jax.numpy as jnp
from jax.experimental import pallas as pl


def kernel(x, edge_index, selections, W, b):
    raise NotImplementedError("write your pallas kernel here")


# speedup vs baseline: 19.9023x; 19.9023x over previous
import functools

import jax
import jax.numpy as jnp
from jax import lax
from jax.experimental import pallas as pl
from jax.experimental.pallas import tpu as pltpu
from jax.experimental.pallas import tpu_sc as plsc

N = 10000
E = 320000
C_IN = 128
C_OUT = 128
K2 = 9

NC = 2
NS = 16
NW = NC * NS
EPW = E // NW
CH = 80
NCHUNK = EPW // CH
VPC = CH // 16
ROWS_PT = 624
TAIL0 = NS * ROWS_PT
TAILR = N - TAIL0


def _xs_body(x_ref, w_ref, xs_ref):
    xs_ref[0] = jnp.dot(x_ref[...], w_ref[0], preferred_element_type=jnp.float32)


def _transform(x, W):
    BN = 2000
    return pl.pallas_call(
        _xs_body,
        grid=(N // BN, K2),
        in_specs=[
            pl.BlockSpec((BN, C_IN), lambda i, s: (i, 0)),
            pl.BlockSpec((1, C_IN, C_OUT), lambda i, s: (s, 0, 0)),
        ],
        out_specs=pl.BlockSpec((1, BN, C_OUT), lambda i, s: (s, i, 0)),
        out_shape=jax.ShapeDtypeStruct((K2, N, C_OUT), jnp.float32),
    )(x, W)


def _prep_body(sel_ref, src_ref, gidx_ref):
    gidx_ref[...] = sel_ref[...] * N + src_ref[...]


def _prep_gidx(selections, src):
    return pl.pallas_call(
        _prep_body,
        out_shape=jax.ShapeDtypeStruct((E // 128, 128), jnp.int32),
    )(selections.reshape(E // 128, 128), src.reshape(E // 128, 128))


def _sc_body(xs_hbm, gidx_hbm, dst_hbm, zeros_hbm, out_hbm,
             gidx_v, dst_v, rows_v, acc, sem):
    c = lax.axis_index("c")
    s = lax.axis_index("s")
    wid = c * NS + s

    pltpu.sync_copy(gidx_hbm.at[wid], gidx_v)
    pltpu.sync_copy(dst_hbm.at[wid], dst_v)
    row0 = s * ROWS_PT
    pltpu.sync_copy(zeros_hbm.at[pl.ds(row0, ROWS_PT)],
                    acc.at[pl.ds(row0, ROWS_PT)])

    @pl.when(s == NS - 1)
    def _():
        pltpu.sync_copy(zeros_hbm.at[pl.ds(TAIL0, TAILR)],
                        acc.at[pl.ds(TAIL0, TAILR)])

    plsc.subcore_barrier()

    def chunk_body(ch, _):
        pltpu.async_copy(xs_hbm.at[gidx_v.at[ch]], rows_v, sem).wait()
        pltpu.sync_copy(rows_v, acc.at[dst_v.at[ch]], add=True)
        return 0
    lax.fori_loop(0, NCHUNK, chunk_body, 0)

    plsc.subcore_barrier()
    pltpu.sync_copy(acc.at[pl.ds(row0, ROWS_PT)],
                    out_hbm.at[c, pl.ds(row0, ROWS_PT)])

    @pl.when(s == NS - 1)
    def _():
        pltpu.sync_copy(acc.at[pl.ds(TAIL0, TAILR)],
                        out_hbm.at[c, pl.ds(TAIL0, TAILR)])


def _scatter(xs2d, gidx3, dst3, zeros):
    mesh = plsc.VectorSubcoreMesh(core_axis_name="c", subcore_axis_name="s",
                                  num_cores=NC, num_subcores=NS)
    fn = pl.kernel(
        _sc_body,
        out_type=jax.ShapeDtypeStruct((NC, N, C_OUT), jnp.float32),
        mesh=mesh,
        scratch_types=[
            pltpu.VMEM((NCHUNK, CH), jnp.int32),
            pltpu.VMEM((NCHUNK, CH), jnp.int32),
            pltpu.VMEM((CH, C_OUT), jnp.float32),
            pltpu.VMEM_SHARED((N, C_OUT), jnp.float32),
            pltpu.SemaphoreType.DMA,
        ],
    )
    return fn(xs2d, gidx3, dst3, zeros)


def _combine_body(p_ref, b_ref, o_ref):
    o_ref[...] = p_ref[0] + p_ref[1] + b_ref[0]


def _combine(partials, b):
    BN = 2000
    return pl.pallas_call(
        _combine_body,
        grid=(N // BN,),
        in_specs=[
            pl.BlockSpec((NC, BN, C_OUT), lambda i: (0, i, 0)),
            pl.BlockSpec((1, C_OUT), lambda i: (0, 0)),
        ],
        out_specs=pl.BlockSpec((BN, C_OUT), lambda i: (i, 0)),
        out_shape=jax.ShapeDtypeStruct((N, C_OUT), jnp.float32),
    )(partials, b.reshape(1, C_OUT))


def kernel(x, edge_index, selections, W, b):
    xs = _transform(x, W).reshape(K2 * N, C_OUT)
    gidx3 = _prep_gidx(selections.astype(jnp.int32),
                       edge_index[0].astype(jnp.int32)).reshape(NW, NCHUNK, CH)
    dst3 = edge_index[1].astype(jnp.int32).reshape(NW, NCHUNK, CH)
    zeros = jnp.zeros((N, C_OUT), jnp.float32)
    partials = _scatter(xs, gidx3, dst3, zeros)
    return _combine(partials, b)


# speedup vs baseline: 29.3875x; 1.4766x over previous
import functools

import jax
import jax.numpy as jnp
from jax import lax
from jax.experimental import pallas as pl
from jax.experimental.pallas import tpu as pltpu
from jax.experimental.pallas import tpu_sc as plsc

N = 10000
E = 320000
C_IN = 128
C_OUT = 128
K2 = 9

NC = 2
NS = 16
NW = NC * NS
EPW = E // NW
CH = 80
NCHUNK = EPW // CH
VPC = CH // 16
ROWS_PT = 624
TAIL0 = NS * ROWS_PT
TAILR = N - TAIL0


def _xs_body(x_ref, w_ref, xs_ref):
    xs_ref[0] = jnp.dot(x_ref[...], w_ref[0], preferred_element_type=jnp.float32)


def _transform(x, W):
    BN = 2000
    return pl.pallas_call(
        _xs_body,
        grid=(N // BN, K2),
        in_specs=[
            pl.BlockSpec((BN, C_IN), lambda i, s: (i, 0)),
            pl.BlockSpec((1, C_IN, C_OUT), lambda i, s: (s, 0, 0)),
        ],
        out_specs=pl.BlockSpec((1, BN, C_OUT), lambda i, s: (s, i, 0)),
        out_shape=jax.ShapeDtypeStruct((K2, N, C_OUT), jnp.float32),
    )(x, W)


def _prep_body(sel_ref, src_ref, dst_ref, pk_ref):
    pk_ref[...] = ((sel_ref[...] * N + src_ref[...]) << 14) | dst_ref[...]


def _prep_pack(selections, src, dst):
    shp = (E // 128, 128)
    return pl.pallas_call(
        _prep_body,
        out_shape=jax.ShapeDtypeStruct(shp, jnp.int32),
    )(selections.reshape(shp), src.reshape(shp), dst.reshape(shp))


def _sc_body(xs_hbm, pk_hbm, zeros_hbm, out_hbm,
             pk_v, gidx_v, dst_v, rows0, rows1, acc, sem0, sem1):
    c = lax.axis_index("c")
    s = lax.axis_index("s")
    wid = c * NS + s

    pltpu.sync_copy(pk_hbm.at[wid], pk_v)
    row0 = s * ROWS_PT
    pltpu.sync_copy(zeros_hbm.at[pl.ds(row0, ROWS_PT)],
                    acc.at[pl.ds(row0, ROWS_PT)])

    @pl.when(s == NS - 1)
    def _():
        pltpu.sync_copy(zeros_hbm.at[pl.ds(TAIL0, TAILR)],
                        acc.at[pl.ds(TAIL0, TAILR)])

    @pl.loop(0, NCHUNK)
    def unpack_body(j):
        for m in range(VPC):
            sl = pl.ds(m * 16, 16)
            p = pk_v[j, sl]
            gidx_v[j, sl] = p >> 14
            dst_v[j, sl] = p & 16383

    plsc.subcore_barrier()

    pltpu.async_copy(xs_hbm.at[gidx_v.at[0]], rows0, sem0)

    @pl.loop(0, NCHUNK - 1, step=2)
    def chunk_body(g):
        pltpu.async_copy(xs_hbm.at[gidx_v.at[g + 1]], rows1, sem1)
        pltpu.make_async_copy(xs_hbm.at[gidx_v.at[g]], rows0, sem0).wait()
        pltpu.sync_copy(rows0, acc.at[dst_v.at[g]], add=True)
        pltpu.async_copy(xs_hbm.at[gidx_v.at[g + 2]], rows0, sem0)
        pltpu.make_async_copy(xs_hbm.at[gidx_v.at[g + 1]], rows1, sem1).wait()
        pltpu.sync_copy(rows1, acc.at[dst_v.at[g + 1]], add=True)

    pltpu.make_async_copy(xs_hbm.at[gidx_v.at[NCHUNK - 1]], rows0, sem0).wait()
    pltpu.sync_copy(rows0, acc.at[dst_v.at[NCHUNK - 1]], add=True)

    plsc.subcore_barrier()
    pltpu.sync_copy(acc.at[pl.ds(row0, ROWS_PT)],
                    out_hbm.at[c, pl.ds(row0, ROWS_PT)])

    @pl.when(s == NS - 1)
    def _():
        pltpu.sync_copy(acc.at[pl.ds(TAIL0, TAILR)],
                        out_hbm.at[c, pl.ds(TAIL0, TAILR)])


def _scatter(xs2d, pk3, zeros):
    mesh = plsc.VectorSubcoreMesh(core_axis_name="c", subcore_axis_name="s",
                                  num_cores=NC, num_subcores=NS)
    fn = pl.kernel(
        _sc_body,
        out_type=jax.ShapeDtypeStruct((NC, N, C_OUT), jnp.float32),
        mesh=mesh,
        compiler_params=pltpu.CompilerParams(use_tc_tiling_on_sc=False),
        scratch_types=[
            pltpu.VMEM((NCHUNK, CH), jnp.int32),
            pltpu.VMEM((NCHUNK, CH), jnp.int32),
            pltpu.VMEM((NCHUNK, CH), jnp.int32),
            pltpu.VMEM((CH, C_OUT), jnp.float32),
            pltpu.VMEM((CH, C_OUT), jnp.float32),
            pltpu.VMEM_SHARED((N, C_OUT), jnp.float32),
            pltpu.SemaphoreType.DMA,
            pltpu.SemaphoreType.DMA,
        ],
    )
    return fn(xs2d, pk3, zeros)


def _combine_body(p_ref, b_ref, o_ref):
    o_ref[...] = p_ref[0] + p_ref[1] + b_ref[0]


def _combine(partials, b):
    BN = 2000
    return pl.pallas_call(
        _combine_body,
        grid=(N // BN,),
        in_specs=[
            pl.BlockSpec((NC, BN, C_OUT), lambda i: (0, i, 0)),
            pl.BlockSpec((1, C_OUT), lambda i: (0, 0)),
        ],
        out_specs=pl.BlockSpec((BN, C_OUT), lambda i: (i, 0)),
        out_shape=jax.ShapeDtypeStruct((N, C_OUT), jnp.float32),
    )(partials, b.reshape(1, C_OUT))


def kernel(x, edge_index, selections, W, b):
    xs = _transform(x, W).reshape(K2 * N, C_OUT)
    pk3 = _prep_pack(selections.astype(jnp.int32),
                     edge_index[0].astype(jnp.int32),
                     edge_index[1].astype(jnp.int32)).reshape(NW, NCHUNK, CH)
    zeros = jnp.zeros((N, C_OUT), jnp.float32)
    partials = _scatter(xs, pk3, zeros)
    return _combine(partials, b)
